# baseline (speedup 1.0000x reference)
import functools

import jax
import jax.numpy as jnp
from jax import lax
from jax.experimental import pallas as pl
from jax.experimental.pallas import tpu as pltpu
from jax.experimental.pallas import tpu_sc as plsc

D = 32
CHUNK = 128


def _build(B):
    info = plsc.get_sparse_core_info()
    NC, NS, L = info.num_cores, info.num_subcores, info.num_lanes
    NW = NC * NS
    bpw = B // NW
    n_chunks = bpw // CHUNK
    mesh = plsc.VectorSubcoreMesh(core_axis_name="c", subcore_axis_name="s")

    @functools.partial(
        pl.kernel,
        mesh=mesh,
        compiler_params=pltpu.CompilerParams(
            needs_layout_passes=False, use_tc_tiling_on_sc=False),
        out_type=jax.ShapeDtypeStruct((B,), jnp.float32),
        scratch_types=[
            pltpu.VMEM((n_chunks, CHUNK), jnp.int32),
            pltpu.VMEM((n_chunks, CHUNK), jnp.int32),
            pltpu.VMEM((n_chunks, CHUNK), jnp.int32),
            pltpu.VMEM((n_chunks, CHUNK), jnp.int32),
            pltpu.VMEM((bpw, D), jnp.float32),
            pltpu.VMEM((bpw, D), jnp.float32),
            pltpu.VMEM((bpw, D), jnp.float32),
            pltpu.VMEM((bpw, D), jnp.float32),
            pltpu.VMEM((bpw,), jnp.float32),
            pltpu.VMEM((bpw,), jnp.float32),
            pltpu.VMEM((bpw,), jnp.float32),
            pltpu.VMEM((bpw,), jnp.float32),
            pltpu.VMEM((bpw,), jnp.float32),
            pltpu.SemaphoreType.DMA,
        ],
    )
    def k(u_hbm, i_hbm, ua_hbm, ia_hbm,
          ue_hbm, ie_hbm, uae_hbm, iae_hbm,
          ub_hbm, ib_hbm, uab_hbm, iab_hbm,
          out_hbm,
          ui_v, ii_v, uai_v, iai_v,
          ue_v, ie_v, uae_v, iae_v,
          ub_v, ib_v, uab_v, iab_v,
          res_v, sem):
        wid = lax.axis_index("s") * NC + lax.axis_index("c")
        base = wid * bpw

        for c in range(n_chunks):
            off = base + c * CHUNK
            pltpu.sync_copy(u_hbm.at[pl.ds(off, CHUNK)], ui_v.at[c])
            pltpu.sync_copy(i_hbm.at[pl.ds(off, CHUNK)], ii_v.at[c])
            pltpu.sync_copy(ua_hbm.at[pl.ds(off, CHUNK)], uai_v.at[c])
            pltpu.sync_copy(ia_hbm.at[pl.ds(off, CHUNK)], iai_v.at[c])

        copies = []
        for c in range(n_chunks):
            rows = pl.ds(c * CHUNK, CHUNK)
            copies += [
                pltpu.async_copy(ue_hbm.at[ui_v.at[c]], ue_v.at[rows], sem),
                pltpu.async_copy(ie_hbm.at[ii_v.at[c]], ie_v.at[rows], sem),
                pltpu.async_copy(uae_hbm.at[uai_v.at[c]], uae_v.at[rows], sem),
                pltpu.async_copy(iae_hbm.at[iai_v.at[c]], iae_v.at[rows], sem),
                pltpu.async_copy(ub_hbm.at[ui_v.at[c]], ub_v.at[rows], sem),
                pltpu.async_copy(ib_hbm.at[ii_v.at[c]], ib_v.at[rows], sem),
                pltpu.async_copy(uab_hbm.at[uai_v.at[c]], uab_v.at[rows], sem),
                pltpu.async_copy(iab_hbm.at[iai_v.at[c]], iab_v.at[rows], sem),
            ]
        for cp in copies:
            cp.wait()

        def body(g, carry):
            b = g * L
            acc = ub_v[pl.ds(b, L)] + ib_v[pl.ds(b, L)]
            acc = acc + uab_v[pl.ds(b, L)] + iab_v[pl.ds(b, L)]
            row = b + lax.iota(jnp.int32, L)
            for d in range(D):
                col = jnp.full((L,), d, jnp.int32)
                ue = plsc.load_gather(ue_v, [row, col])
                ie = plsc.load_gather(ie_v, [row, col])
                uae = plsc.load_gather(uae_v, [row, col])
                iae = plsc.load_gather(iae_v, [row, col])
                acc = acc + ue * (ie + uae) + ie * iae
            res_v[pl.ds(b, L)] = acc
            return carry

        lax.fori_loop(0, bpw // L, body, 0)
        pltpu.sync_copy(res_v, out_hbm.at[pl.ds(base, bpw)])

    return k


def kernel(user, item, user_attributes, item_attributes, user_emb, item_emb,
           user_attr_emb, item_attr_emb, user_bias, item_bias,
           user_attr_bias, item_attr_bias):
    B = user.shape[0]
    k = _build(B)
    return k(user, item, user_attributes, item_attributes,
             user_emb, item_emb, user_attr_emb, item_attr_emb,
             user_bias.reshape(-1), item_bias.reshape(-1),
             user_attr_bias.reshape(-1), item_attr_bias.reshape(-1))


# speedup vs baseline: 1.2171x; 1.2171x over previous
import functools

import jax
import jax.numpy as jnp
from jax import lax
from jax.experimental import pallas as pl
from jax.experimental.pallas import tpu as pltpu
from jax.experimental.pallas import tpu_sc as plsc

D = 32
TG = 4
TR = 8
CHUNK = 128
CE = 16
W = 16


def _build(B):
    info = plsc.get_sparse_core_info()
    NC, NS, L = info.num_cores, info.num_subcores, info.num_lanes
    NW = NC * NS
    bpw = B // NW
    n_chunks = bpw // CHUNK
    n_ce = bpw // CE
    half = CE * W
    mesh = plsc.VectorSubcoreMesh(core_axis_name="c", subcore_axis_name="s")

    @functools.partial(
        pl.kernel,
        mesh=mesh,
        compiler_params=pltpu.CompilerParams(needs_layout_passes=False),
        out_type=jax.ShapeDtypeStruct((B,), jnp.float32),
        scratch_types=[
            pltpu.VMEM((n_chunks, CHUNK), jnp.int32),
            pltpu.VMEM((n_chunks, CHUNK), jnp.int32),
            pltpu.VMEM((n_chunks, CHUNK), jnp.int32),
            pltpu.VMEM((n_chunks, CHUNK), jnp.int32),
            pltpu.VMEM((bpw,), jnp.int32),
            pltpu.VMEM((bpw,), jnp.int32),
            pltpu.VMEM((bpw,), jnp.int32),
            pltpu.VMEM((bpw,), jnp.int32),
            pltpu.VMEM((TG, TR, 2 * half), jnp.float32),
            pltpu.VMEM((TG, TR, 2 * half), jnp.float32),
            pltpu.VMEM((TG, TR, 2 * half), jnp.float32),
            pltpu.VMEM((TG, TR, 2 * half), jnp.float32),
            pltpu.VMEM((bpw,), jnp.float32),
            pltpu.VMEM((bpw,), jnp.float32),
            pltpu.VMEM((bpw,), jnp.float32),
            pltpu.VMEM((bpw,), jnp.float32),
            pltpu.VMEM((bpw,), jnp.float32),
            pltpu.VMEM((4, TR, 2 * L), jnp.int32),
            pltpu.VMEM((4, TR, 2 * L), jnp.int32),
            pltpu.SemaphoreType.DMA,
            pltpu.SemaphoreType.DMA,
        ],
    )
    def k(u_hbm, i_hbm, ua_hbm, ia_hbm,
          ue_hbm, ie_hbm, uae_hbm, iae_hbm,
          ub_hbm, ib_hbm, uab_hbm, iab_hbm,
          out_hbm,
          uic_v, iic_v, uaic_v, iaic_v,
          uif_v, iif_v, uaif_v, iaif_v,
          ue_v, ie_v, uae_v, iae_v,
          ub_v, ib_v, uab_v, iab_v,
          res_v, bki_v, bke_v, sem_e, sem_b):
        wid = lax.axis_index("s") * NC + lax.axis_index("c")
        base = wid * bpw

        emb_tbls = (ue_hbm, ie_hbm, uae_hbm, iae_hbm)
        emb_dsts = (ue_v, ie_v, uae_v, iae_v)
        idx_flats = (uif_v, iif_v, uaif_v, iaif_v)

        for c in range(n_chunks):
            off = base + c * CHUNK
            pltpu.sync_copy(u_hbm.at[pl.ds(off, CHUNK)], uic_v.at[c])
            pltpu.sync_copy(i_hbm.at[pl.ds(off, CHUNK)], iic_v.at[c])
            pltpu.sync_copy(ua_hbm.at[pl.ds(off, CHUNK)], uaic_v.at[c])
            pltpu.sync_copy(ia_hbm.at[pl.ds(off, CHUNK)], iaic_v.at[c])
        pltpu.sync_copy(u_hbm.at[pl.ds(base, bpw)], uif_v)
        pltpu.sync_copy(i_hbm.at[pl.ds(base, bpw)], iif_v)
        pltpu.sync_copy(ua_hbm.at[pl.ds(base, bpw)], uaif_v)
        pltpu.sync_copy(ia_hbm.at[pl.ds(base, bpw)], iaif_v)

        for c in range(n_chunks):
            rows = pl.ds(c * CHUNK, CHUNK)
            pltpu.async_copy(ub_hbm.at[uic_v.at[c]], ub_v.at[rows], sem_b)
            pltpu.async_copy(ib_hbm.at[iic_v.at[c]], ib_v.at[rows], sem_b)
            pltpu.async_copy(uab_hbm.at[uaic_v.at[c]], uab_v.at[rows], sem_b)
            pltpu.async_copy(iab_hbm.at[iaic_v.at[c]], iab_v.at[rows], sem_b)
        for bias_v, bias_hbm in ((ub_v, ub_hbm), (ib_v, ib_hbm),
                                 (uab_v, uab_hbm), (iab_v, iab_hbm)):
            pltpu.make_async_copy(bias_hbm.at[pl.ds(0, bpw)], bias_v,
                                  sem_b).wait()

        iota_l = lax.iota(jnp.int32, L)

        def issue(cc, pb):
            dbase = pb * half
            for ti, (tbl, idxf, dst) in enumerate(
                    zip(emb_tbls, idx_flats, emb_dsts)):
                vec = idxf[pl.ds(cc * CE, CE)]
                kv = (vec >> 4) & 7
                counts = []
                for kk in range(TR):
                    mask = kv == kk
                    cnt = plsc.all_reduce_population_count(mask)
                    plsc.store_compressed(bki_v.at[ti, kk, pl.ds(0, L)],
                                          vec, mask=mask)
                    plsc.store_compressed(bke_v.at[ti, kk, pl.ds(0, L)],
                                          iota_l, mask=mask)
                    counts.append(cnt)
                for kk in range(TR):
                    cnt = counts[kk]
                    cnt0 = cnt if cnt.ndim == 0 else cnt[0]

                    def ibody(p, carry, ti=ti, kk=kk):
                        iv = bki_v[ti, kk, pl.ds(p, L)]
                        ev = bke_v[ti, kk, pl.ds(p, L)]
                        i = iv[0]
                        e = ev[0]
                        bb = pl.multiple_of((i >> 7) * 128, 128)
                        pltpu.async_copy(
                            tbl.at[:, :, pl.ds(bb + kk * W, W)],
                            dst.at[:, :, pl.ds(dbase + e * W, W)],
                            sem_e)
                        return carry
                    lax.fori_loop(0, cnt0, ibody, 0)

        def drain():
            for tbl, dst in zip(emb_tbls, emb_dsts):
                pltpu.make_async_copy(tbl.at[:, :, pl.ds(0, half)],
                                      dst.at[:, :, pl.ds(0, half)],
                                      sem_e).wait()

        iota16 = lax.iota(jnp.int32, L) * W

        def compute(c, pb):
            e0 = c * CE
            sl = pl.ds(e0, CE)
            acc = ub_v[sl] + ib_v[sl]
            acc = acc + uab_v[sl] + iab_v[sl]
            dbase = pb * half
            cols = []
            for idxf in idx_flats:
                lv = idxf[sl] & (W - 1)
                cols.append(dbase + iota16 + lv)
            cu, ci, cua, cia = cols
            for d in range(D):
                t, r = d // TR, d % TR
                tv = jnp.full((L,), t, jnp.int32)
                rv = jnp.full((L,), r, jnp.int32)
                ue = plsc.load_gather(ue_v, [tv, rv, cu])
                ie = plsc.load_gather(ie_v, [tv, rv, ci])
                uae = plsc.load_gather(uae_v, [tv, rv, cua])
                iae = plsc.load_gather(iae_v, [tv, rv, cia])
                acc = acc + ue * (ie + uae) + ie * iae
            res_v[sl] = acc

        def body(c, carry):
            @pl.when(c > 0)
            def _():
                drain()

            @pl.when(c < n_ce)
            def _():
                issue(c, c & 1)

            @pl.when(c > 0)
            def _():
                compute(c - 1, (c - 1) & 1)
            return carry

        lax.fori_loop(0, n_ce + 1, body, 0)
        pltpu.sync_copy(res_v, out_hbm.at[pl.ds(base, bpw)])

    return k


def kernel(user, item, user_attributes, item_attributes, user_emb, item_emb,
           user_attr_emb, item_attr_emb, user_bias, item_bias,
           user_attr_bias, item_attr_bias):
    B = user.shape[0]
    n_user, n_item = user_emb.shape[0], item_emb.shape[0]
    n_ua, n_ia = user_attr_emb.shape[0], item_attr_emb.shape[0]
    k = _build(B)
    return k(user, item, user_attributes, item_attributes,
             user_emb.T.reshape(TG, TR, n_user),
             item_emb.T.reshape(TG, TR, n_item),
             user_attr_emb.T.reshape(TG, TR, n_ua),
             item_attr_emb.T.reshape(TG, TR, n_ia),
             user_bias.reshape(-1), item_bias.reshape(-1),
             user_attr_bias.reshape(-1), item_attr_bias.reshape(-1))


# speedup vs baseline: 1.8943x; 1.5564x over previous
import functools

import jax
import jax.numpy as jnp
from jax import lax
from jax.experimental import pallas as pl
from jax.experimental.pallas import tpu as pltpu
from jax.experimental.pallas import tpu_sc as plsc

D = 32
TG = 4
TR = 8
CHUNK = 128
CE = 16
W = 16


def _build(B):
    info = plsc.get_sparse_core_info()
    NC, NS, L = info.num_cores, info.num_subcores, info.num_lanes
    NW = NC * NS
    bpw = B // NW
    n_chunks = bpw // CHUNK
    n_ce = bpw // CE
    half = CE * W
    n_arows = bpw * D // CHUNK
    mesh = plsc.VectorSubcoreMesh(core_axis_name="c", subcore_axis_name="s")

    @functools.partial(
        pl.kernel,
        mesh=mesh,
        compiler_params=pltpu.CompilerParams(needs_layout_passes=False),
        out_type=jax.ShapeDtypeStruct((B,), jnp.float32),
        scratch_types=[
            pltpu.VMEM((n_chunks, CHUNK), jnp.int32),
            pltpu.VMEM((n_chunks, CHUNK), jnp.int32),
            pltpu.VMEM((n_chunks, CHUNK), jnp.int32),
            pltpu.VMEM((n_chunks, CHUNK), jnp.int32),
            pltpu.VMEM((bpw,), jnp.int32),
            pltpu.VMEM((bpw,), jnp.int32),
            pltpu.VMEM((bpw,), jnp.int32),
            pltpu.VMEM((bpw,), jnp.int32),
            pltpu.VMEM((TG, TR, 2 * half), jnp.float32),
            pltpu.VMEM((TG, TR, 2 * half), jnp.float32),
            pltpu.VMEM((n_arows, CHUNK), jnp.int32),
            pltpu.VMEM((n_arows, CHUNK), jnp.int32),
            pltpu.VMEM((bpw * D,), jnp.float32),
            pltpu.VMEM((bpw * D,), jnp.float32),
            pltpu.VMEM((bpw,), jnp.float32),
            pltpu.VMEM((bpw,), jnp.float32),
            pltpu.VMEM((bpw,), jnp.float32),
            pltpu.VMEM((bpw,), jnp.float32),
            pltpu.VMEM((bpw,), jnp.float32),
            pltpu.VMEM((2, TR, 2 * L), jnp.int32),
            pltpu.VMEM((2, TR, 2 * L), jnp.int32),
            pltpu.SemaphoreType.DMA,
            pltpu.SemaphoreType.DMA,
        ],
    )
    def k(u_hbm, i_hbm, ua_hbm, ia_hbm,
          ue_hbm, ie_hbm, uae_hbm, iae_hbm,
          ub_hbm, ib_hbm, uab_hbm, iab_hbm,
          out_hbm,
          uic_v, iic_v, uaic_v, iaic_v,
          uif_v, iif_v, uaif_v, iaif_v,
          ue_v, ie_v,
          uaidx_v, iaidx_v, uaval_v, iaval_v,
          ub_v, ib_v, uab_v, iab_v,
          res_v, bki_v, bke_v, sem_e, sem_b):
        wid = lax.axis_index("s") * NC + lax.axis_index("c")
        base = wid * bpw

        emb_tbls = (ue_hbm, ie_hbm)
        emb_dsts = (ue_v, ie_v)
        idx_flats = (uif_v, iif_v)

        for c in range(n_chunks):
            off = base + c * CHUNK
            pltpu.sync_copy(u_hbm.at[pl.ds(off, CHUNK)], uic_v.at[c])
            pltpu.sync_copy(i_hbm.at[pl.ds(off, CHUNK)], iic_v.at[c])
            pltpu.sync_copy(ua_hbm.at[pl.ds(off, CHUNK)], uaic_v.at[c])
            pltpu.sync_copy(ia_hbm.at[pl.ds(off, CHUNK)], iaic_v.at[c])
        pltpu.sync_copy(u_hbm.at[pl.ds(base, bpw)], uif_v)
        pltpu.sync_copy(i_hbm.at[pl.ds(base, bpw)], iif_v)
        pltpu.sync_copy(ua_hbm.at[pl.ds(base, bpw)], uaif_v)
        pltpu.sync_copy(ia_hbm.at[pl.ds(base, bpw)], iaif_v)

        for c in range(n_chunks):
            rows = pl.ds(c * CHUNK, CHUNK)
            pltpu.async_copy(ub_hbm.at[uic_v.at[c]], ub_v.at[rows], sem_b)
            pltpu.async_copy(ib_hbm.at[iic_v.at[c]], ib_v.at[rows], sem_b)
            pltpu.async_copy(uab_hbm.at[uaic_v.at[c]], uab_v.at[rows], sem_b)
            pltpu.async_copy(iab_hbm.at[iaic_v.at[c]], iab_v.at[rows], sem_b)

        def abuild(g, carry):
            ua = uaif_v[pl.ds(g * L, L)] * D
            ia = iaif_v[pl.ds(g * L, L)] * D
            for d in range(D):
                pos = d * bpw + g * L
                r = d * (bpw // CHUNK) + (g >> 3)
                col = (g & 7) * L
                uaidx_v[r, pl.ds(col, L)] = ua + d
                iaidx_v[r, pl.ds(col, L)] = ia + d
            return carry

        lax.fori_loop(0, bpw // L, abuild, 0)

        def afire(r, carry):
            pltpu.async_copy(uae_hbm.at[uaidx_v.at[r]],
                             uaval_v.at[pl.ds(r * CHUNK, CHUNK)], sem_b)
            pltpu.async_copy(iae_hbm.at[iaidx_v.at[r]],
                             iaval_v.at[pl.ds(r * CHUNK, CHUNK)], sem_b)
            return carry

        lax.fori_loop(0, n_arows, afire, 0)

        iota_l = lax.iota(jnp.int32, L)

        def issue(cc, pb):
            dbase = pb * half
            for ti, (tbl, idxf, dst) in enumerate(
                    zip(emb_tbls, idx_flats, emb_dsts)):
                vec = idxf[pl.ds(cc * CE, CE)]
                kv = (vec >> 4) & 7
                counts = []
                for kk in range(TR):
                    mask = kv == kk
                    cnt = plsc.all_reduce_population_count(mask)
                    plsc.store_compressed(bki_v.at[ti, kk, pl.ds(0, L)],
                                          vec, mask=mask)
                    plsc.store_compressed(bke_v.at[ti, kk, pl.ds(0, L)],
                                          iota_l, mask=mask)
                    counts.append(cnt)
                for kk in range(TR):
                    cnt = counts[kk]
                    cnt0 = cnt if cnt.ndim == 0 else cnt[0]

                    def ibody(p, carry, ti=ti, kk=kk):
                        iv = bki_v[ti, kk, pl.ds(p, L)]
                        ev = bke_v[ti, kk, pl.ds(p, L)]
                        i = iv[0]
                        e = ev[0]
                        bb = pl.multiple_of((i >> 7) * 128, 128)
                        pltpu.async_copy(
                            tbl.at[:, :, pl.ds(bb + kk * W, W)],
                            dst.at[:, :, pl.ds(dbase + e * W, W)],
                            sem_e)
                        return carry
                    lax.fori_loop(0, cnt0, ibody, 0)

        def drain():
            for tbl, dst in zip(emb_tbls, emb_dsts):
                pltpu.make_async_copy(tbl.at[:, :, pl.ds(0, half)],
                                      dst.at[:, :, pl.ds(0, half)],
                                      sem_e).wait()

        iota16 = lax.iota(jnp.int32, L) * W

        def compute(c, pb):
            e0 = c * CE
            sl = pl.ds(e0, CE)
            acc = ub_v[sl] + ib_v[sl]
            acc = acc + uab_v[sl] + iab_v[sl]
            dbase = pb * half
            cu = dbase + iota16 + (uif_v[sl] & (W - 1))
            ci = dbase + iota16 + (iif_v[sl] & (W - 1))
            for d in range(D):
                t, r = d // TR, d % TR
                tv = jnp.full((L,), t, jnp.int32)
                rv = jnp.full((L,), r, jnp.int32)
                ue = plsc.load_gather(ue_v, [tv, rv, cu])
                ie = plsc.load_gather(ie_v, [tv, rv, ci])
                uae = uaval_v[pl.ds(d * bpw + e0, L)]
                iae = iaval_v[pl.ds(d * bpw + e0, L)]
                acc = acc + ue * (ie + uae) + ie * iae
            res_v[sl] = acc

        for bias_v, bias_hbm in ((ub_v, ub_hbm), (ib_v, ib_hbm),
                                 (uab_v, uab_hbm), (iab_v, iab_hbm)):
            pltpu.make_async_copy(bias_hbm.at[pl.ds(0, bpw)], bias_v,
                                  sem_b).wait()
        pltpu.make_async_copy(uae_hbm.at[pl.ds(0, bpw * D)], uaval_v,
                              sem_b).wait()
        pltpu.make_async_copy(iae_hbm.at[pl.ds(0, bpw * D)], iaval_v,
                              sem_b).wait()

        def body(c, carry):
            @pl.when(c > 0)
            def _():
                drain()

            @pl.when(c < n_ce)
            def _():
                issue(c, c & 1)

            @pl.when(c > 0)
            def _():
                compute(c - 1, (c - 1) & 1)
            return carry

        lax.fori_loop(0, n_ce + 1, body, 0)
        pltpu.sync_copy(res_v, out_hbm.at[pl.ds(base, bpw)])

    return k


def kernel(user, item, user_attributes, item_attributes, user_emb, item_emb,
           user_attr_emb, item_attr_emb, user_bias, item_bias,
           user_attr_bias, item_attr_bias):
    B = user.shape[0]
    n_user, n_item = user_emb.shape[0], item_emb.shape[0]
    k = _build(B)
    return k(user, item, user_attributes, item_attributes,
             user_emb.T.reshape(TG, TR, n_user),
             item_emb.T.reshape(TG, TR, n_item),
             user_attr_emb.reshape(-1), item_attr_emb.reshape(-1),
             user_bias.reshape(-1), item_bias.reshape(-1),
             user_attr_bias.reshape(-1), item_attr_bias.reshape(-1))
